# baseline (device time: 23841 ns/iter reference)
import jax
import jax.numpy as jnp
from jax import lax
from jax.experimental import pallas as pl
from jax.experimental.pallas import tpu as pltpu

N_DEV = 16
N_TOK = 512
D_IN = 256
D_OUT = 512
EXP_PER_DEV = 2
CAP = 12
ROWS_PER_DEV = N_TOK // N_DEV

_DEVICE_ID_TYPE = getattr(pl, "DeviceIdType", None) or pltpu.DeviceIdType


def kernel(x, router_W, route_idx, expert_W):
    del router_W

    def body(x_ref, route_ref, ew_ref, out_ref,
             p_ref, recv_ref, send_sems, recv_sems):
        d = lax.axis_index("i")

        r = route_ref[...]
        r_cols = r.reshape(1, N_TOK)
        row_i = lax.broadcasted_iota(jnp.int32, (N_TOK, N_TOK), 0)
        col_j = lax.broadcasted_iota(jnp.int32, (N_TOK, N_TOK), 1)
        same = (r == r_cols) & (col_j <= row_i)
        rank = jnp.sum(same.astype(jnp.int32), axis=1, keepdims=True)
        keep = rank <= CAP

        e0 = EXP_PER_DEV * d
        sel0 = (keep & (r == e0)).astype(jnp.float32)
        sel1 = (keep & (r == e0 + 1)).astype(jnp.float32)

        xv = x_ref[...]
        p_ref[...] = (
            sel0 * jnp.dot(xv, ew_ref[0], preferred_element_type=jnp.float32)
            + sel1 * jnp.dot(xv, ew_ref[1], preferred_element_type=jnp.float32)
        )

        recv_ref[0] = p_ref[pl.ds(d * ROWS_PER_DEV, ROWS_PER_DEV), :]

        rdmas = []
        for o in range(1, N_DEV):
            tgt = lax.rem(d + o, N_DEV)
            rdma = pltpu.make_async_remote_copy(
                src_ref=p_ref.at[pl.ds(tgt * ROWS_PER_DEV, ROWS_PER_DEV), :],
                dst_ref=recv_ref.at[o],
                send_sem=send_sems.at[o],
                recv_sem=recv_sems.at[o],
                device_id=(tgt,),
                device_id_type=_DEVICE_ID_TYPE.MESH,
            )
            rdma.start()
            rdmas.append(rdma)
        for rdma in rdmas:
            rdma.wait()

        out_ref[...] = jnp.sum(recv_ref[...], axis=0)

    return pl.pallas_call(
        body,
        out_shape=jax.ShapeDtypeStruct((ROWS_PER_DEV, D_OUT), jnp.float32),
        in_specs=[
            pl.BlockSpec(memory_space=pltpu.VMEM),
            pl.BlockSpec(memory_space=pltpu.VMEM),
            pl.BlockSpec(memory_space=pltpu.VMEM),
        ],
        out_specs=pl.BlockSpec(memory_space=pltpu.VMEM),
        scratch_shapes=[
            pltpu.VMEM((N_TOK, D_OUT), jnp.float32),
            pltpu.VMEM((N_DEV, ROWS_PER_DEV, D_OUT), jnp.float32),
            pltpu.SemaphoreType.DMA((N_DEV,)),
            pltpu.SemaphoreType.DMA((N_DEV,)),
        ],
    )(x, route_idx, expert_W)
